# initial kernel scaffold (unmeasured)
import jax
import jax.numpy as jnp
from jax import lax
from jax.experimental import pallas as pl
from jax.experimental.pallas import tpu as pltpu

N_DEV = 4


def kernel(x, router, W1, W2):
    t_per, d_model = x.shape
    e_per, _, f_dim = W1.shape
    e_shard = router.shape[1]
    n_exp = N_DEV * e_shard
    t_tot = N_DEV * t_per

    def body(x_ref, r_ref, w1_ref, w2_ref, out_ref,
             x_all, r_all, p_buf, c_buf,
             xs_s, xs_r, rs_s, rs_r, cs_s, cs_r):
        my = lax.axis_index("i")

        barrier = pltpu.get_barrier_semaphore()
        for d in (1, 2, 3):
            pl.semaphore_signal(
                barrier, inc=1, device_id=((my + d) % N_DEV,),
                device_id_type=pl.DeviceIdType.MESH)
        pl.semaphore_wait(barrier, N_DEV - 1)

        x_all[my, :, :] = x_ref[...]
        r_all[my, :, :] = r_ref[...]

        x_sends = []
        r_sends = []
        for d in (1, 2, 3):
            peer = (my + d) % N_DEV
            xr = pltpu.make_async_remote_copy(
                src_ref=x_all.at[my], dst_ref=x_all.at[my],
                send_sem=xs_s.at[d], recv_sem=xs_r.at[d],
                device_id=(peer,), device_id_type=pl.DeviceIdType.MESH)
            xr.start()
            x_sends.append(xr)
            rr = pltpu.make_async_remote_copy(
                src_ref=r_all.at[my], dst_ref=r_all.at[my],
                send_sem=rs_s.at[d], recv_sem=rs_r.at[d],
                device_id=(peer,), device_id_type=pl.DeviceIdType.MESH)
            rr.start()
            r_sends.append(rr)

        for d in (1, 2, 3):
            src = (my - d) % N_DEV
            pltpu.make_async_remote_copy(
                src_ref=x_all.at[src], dst_ref=x_all.at[src],
                send_sem=xs_s.at[d], recv_sem=xs_r.at[d],
                device_id=(my,), device_id_type=pl.DeviceIdType.MESH
            ).wait_recv()
            pltpu.make_async_remote_copy(
                src_ref=r_all.at[src], dst_ref=r_all.at[src],
                send_sem=rs_s.at[d], recv_sem=rs_r.at[d],
                device_id=(my,), device_id_type=pl.DeviceIdType.MESH
            ).wait_recv()

        x_full = x_all[...].reshape(t_tot, d_model)
        rv = r_all[...]
        router_full = jnp.concatenate(
            [rv[0], rv[1], rv[2], rv[3]], axis=1)

        gates = jnp.dot(x_full, router_full,
                        preferred_element_type=jnp.float32)
        eiota = lax.broadcasted_iota(jnp.int32, (t_tot, n_exp), 1)
        v1 = jnp.max(gates, axis=1, keepdims=True)
        i1 = jnp.min(jnp.where(gates == v1, eiota, n_exp),
                     axis=1, keepdims=True)
        g2 = jnp.where(eiota == i1, -jnp.inf, gates)
        v2 = jnp.max(g2, axis=1, keepdims=True)
        i2 = jnp.min(jnp.where(g2 == v2, eiota, n_exp),
                     axis=1, keepdims=True)
        z = jnp.exp(v2 - v1)
        w_full = (jnp.where(eiota == i1, 1.0, 0.0)
                  + jnp.where(eiota == i2, z, 0.0)) / (1.0 + z)

        x_bf = x_full.astype(jnp.bfloat16)
        partial = jnp.zeros((t_tot, d_model), jnp.float32)
        for e in range(e_per):
            w1e = w1_ref[e, :, :].astype(jnp.bfloat16)
            h = jnp.dot(x_bf, w1e, preferred_element_type=jnp.float32)
            h = jnp.maximum(h, 0.0).astype(jnp.bfloat16)
            w2e = w2_ref[e, :, :].astype(jnp.bfloat16)
            o = jnp.dot(h, w2e, preferred_element_type=jnp.float32)
            ge = e_shard * my + e
            wsel = jnp.sum(jnp.where(eiota == ge, w_full, 0.0),
                           axis=1, keepdims=True)
            partial = partial + o * wsel

        p_buf[...] = partial.astype(jnp.bfloat16).reshape(
            N_DEV, t_per, d_model)
        c_buf[my, :, :] = p_buf[my, :, :]
        c_sends = []
        for d in (1, 2, 3):
            peer = (my + d) % N_DEV
            cr = pltpu.make_async_remote_copy(
                src_ref=p_buf.at[peer], dst_ref=c_buf.at[my],
                send_sem=cs_s.at[d], recv_sem=cs_r.at[d],
                device_id=(peer,), device_id_type=pl.DeviceIdType.MESH)
            cr.start()
            c_sends.append(cr)
        for d in (1, 2, 3):
            src = (my - d) % N_DEV
            pltpu.make_async_remote_copy(
                src_ref=c_buf.at[src], dst_ref=c_buf.at[src],
                send_sem=cs_s.at[d], recv_sem=cs_r.at[d],
                device_id=(my,), device_id_type=pl.DeviceIdType.MESH
            ).wait_recv()

        cv = c_buf[...]
        out_ref[...] = (cv[0].astype(jnp.float32)
                        + cv[1].astype(jnp.float32)
                        + cv[2].astype(jnp.float32)
                        + cv[3].astype(jnp.float32))

        for rd in x_sends + r_sends + c_sends:
            rd.wait_send()

    return pl.pallas_call(
        body,
        out_shape=jax.ShapeDtypeStruct((t_per, d_model), jnp.float32),
        in_specs=[pl.BlockSpec(memory_space=pltpu.VMEM)] * 4,
        out_specs=pl.BlockSpec(memory_space=pltpu.VMEM),
        scratch_shapes=[
            pltpu.VMEM((N_DEV, t_per, d_model), jnp.float32),
            pltpu.VMEM((N_DEV, t_tot, e_shard), jnp.float32),
            pltpu.VMEM((N_DEV, t_per, d_model), jnp.bfloat16),
            pltpu.VMEM((N_DEV, t_per, d_model), jnp.bfloat16),
            pltpu.SemaphoreType.DMA((N_DEV,)),
            pltpu.SemaphoreType.DMA((N_DEV,)),
            pltpu.SemaphoreType.DMA((N_DEV,)),
            pltpu.SemaphoreType.DMA((N_DEV,)),
            pltpu.SemaphoreType.DMA((N_DEV,)),
            pltpu.SemaphoreType.DMA((N_DEV,)),
        ],
        compiler_params=pltpu.CompilerParams(collective_id=0),
    )(x, router, W1, W2)


# baseline (device time: 31984 ns/iter reference)
import jax
import jax.numpy as jnp
from jax import lax
from jax.experimental import pallas as pl
from jax.experimental.pallas import tpu as pltpu

N_DEV = 4


def kernel(x, router, W1, W2):
    t_per, d_model = x.shape
    e_per, _, f_dim = W1.shape
    e_shard = router.shape[1]
    n_exp = N_DEV * e_shard
    t_tot = N_DEV * t_per

    def body(x_ref, r_ref, w1_ref, w2_ref, out_ref,
             x_all, r_all, p_buf, c_buf,
             xs_s, xs_r, rs_s, rs_r, cs_s, cs_r):
        my = lax.axis_index("i")

        barrier = pltpu.get_barrier_semaphore()
        for d in (1, 2, 3):
            pl.semaphore_signal(
                barrier, inc=1, device_id=((my + d) % N_DEV,),
                device_id_type=pl.DeviceIdType.MESH)
        pl.semaphore_wait(barrier, N_DEV - 1)

        x_all[my, :, :] = x_ref[...]
        r_all[my, :, :] = r_ref[...]

        x_sends = []
        r_sends = []
        for d in (1, 2, 3):
            peer = (my + d) % N_DEV
            xr = pltpu.make_async_remote_copy(
                src_ref=x_all.at[my], dst_ref=x_all.at[my],
                send_sem=xs_s.at[d], recv_sem=xs_r.at[d],
                device_id=(peer,), device_id_type=pl.DeviceIdType.MESH)
            xr.start()
            x_sends.append(xr)
            rr = pltpu.make_async_remote_copy(
                src_ref=r_all.at[my], dst_ref=r_all.at[my],
                send_sem=rs_s.at[d], recv_sem=rs_r.at[d],
                device_id=(peer,), device_id_type=pl.DeviceIdType.MESH)
            rr.start()
            r_sends.append(rr)

        for d in (1, 2, 3):
            src = (my - d) % N_DEV
            pltpu.make_async_remote_copy(
                src_ref=x_all.at[src], dst_ref=x_all.at[src],
                send_sem=xs_s.at[d], recv_sem=xs_r.at[d],
                device_id=(my,), device_id_type=pl.DeviceIdType.MESH
            ).wait_recv()
            pltpu.make_async_remote_copy(
                src_ref=r_all.at[src], dst_ref=r_all.at[src],
                send_sem=rs_s.at[d], recv_sem=rs_r.at[d],
                device_id=(my,), device_id_type=pl.DeviceIdType.MESH
            ).wait_recv()

        x_full = x_all[...].reshape(t_tot, d_model)
        rv = r_all[...]
        router_full = jnp.concatenate(
            [rv[0], rv[1], rv[2], rv[3]], axis=1)

        gates = jnp.dot(x_full, router_full,
                        precision=lax.Precision.HIGHEST,
                        preferred_element_type=jnp.float32)
        eiota = lax.broadcasted_iota(jnp.int32, (t_tot, n_exp), 1)
        v1 = jnp.max(gates, axis=1, keepdims=True)
        i1 = jnp.min(jnp.where(gates == v1, eiota, n_exp),
                     axis=1, keepdims=True)
        g2 = jnp.where(eiota == i1, -jnp.inf, gates)
        v2 = jnp.max(g2, axis=1, keepdims=True)
        i2 = jnp.min(jnp.where(g2 == v2, eiota, n_exp),
                     axis=1, keepdims=True)
        z = jnp.exp(v2 - v1)
        w_full = (jnp.where(eiota == i1, 1.0, 0.0)
                  + jnp.where(eiota == i2, z, 0.0)) / (1.0 + z)

        x_bf = x_full.astype(jnp.bfloat16)
        partial = jnp.zeros((t_tot, d_model), jnp.float32)
        for e in range(e_per):
            w1e = w1_ref[e, :, :].astype(jnp.bfloat16)
            h = jnp.dot(x_bf, w1e, preferred_element_type=jnp.float32)
            h = jnp.maximum(h, 0.0).astype(jnp.bfloat16)
            w2e = w2_ref[e, :, :].astype(jnp.bfloat16)
            o = jnp.dot(h, w2e, preferred_element_type=jnp.float32)
            ge = e_shard * my + e
            wsel = jnp.sum(jnp.where(eiota == ge, w_full, 0.0),
                           axis=1, keepdims=True)
            partial = partial + o * wsel

        p_buf[...] = partial.astype(jnp.bfloat16).reshape(
            N_DEV, t_per, d_model)
        c_buf[my, :, :] = p_buf[my, :, :]
        c_sends = []
        for d in (1, 2, 3):
            peer = (my + d) % N_DEV
            cr = pltpu.make_async_remote_copy(
                src_ref=p_buf.at[peer], dst_ref=c_buf.at[my],
                send_sem=cs_s.at[d], recv_sem=cs_r.at[d],
                device_id=(peer,), device_id_type=pl.DeviceIdType.MESH)
            cr.start()
            c_sends.append(cr)
        for d in (1, 2, 3):
            src = (my - d) % N_DEV
            pltpu.make_async_remote_copy(
                src_ref=c_buf.at[src], dst_ref=c_buf.at[src],
                send_sem=cs_s.at[d], recv_sem=cs_r.at[d],
                device_id=(my,), device_id_type=pl.DeviceIdType.MESH
            ).wait_recv()

        cv = c_buf[...]
        out_ref[...] = (cv[0].astype(jnp.float32)
                        + cv[1].astype(jnp.float32)
                        + cv[2].astype(jnp.float32)
                        + cv[3].astype(jnp.float32))

        for rd in x_sends + r_sends + c_sends:
            rd.wait_send()

    return pl.pallas_call(
        body,
        out_shape=jax.ShapeDtypeStruct((t_per, d_model), jnp.float32),
        in_specs=[pl.BlockSpec(memory_space=pltpu.VMEM)] * 4,
        out_specs=pl.BlockSpec(memory_space=pltpu.VMEM),
        scratch_shapes=[
            pltpu.VMEM((N_DEV, t_per, d_model), jnp.float32),
            pltpu.VMEM((N_DEV, t_tot, e_shard), jnp.float32),
            pltpu.VMEM((N_DEV, t_per, d_model), jnp.bfloat16),
            pltpu.VMEM((N_DEV, t_per, d_model), jnp.bfloat16),
            pltpu.SemaphoreType.DMA((N_DEV,)),
            pltpu.SemaphoreType.DMA((N_DEV,)),
            pltpu.SemaphoreType.DMA((N_DEV,)),
            pltpu.SemaphoreType.DMA((N_DEV,)),
            pltpu.SemaphoreType.DMA((N_DEV,)),
            pltpu.SemaphoreType.DMA((N_DEV,)),
        ],
        compiler_params=pltpu.CompilerParams(collective_id=0),
    )(x, router, W1, W2)


# device time: 30784 ns/iter; 1.0390x vs baseline; 1.0390x over previous
import jax
import jax.numpy as jnp
from jax import lax
from jax.experimental import pallas as pl
from jax.experimental.pallas import tpu as pltpu

N_DEV = 4


def kernel(x, router, W1, W2):
    t_per, d_model = x.shape
    e_per, _, f_dim = W1.shape
    e_shard = router.shape[1]
    n_exp = N_DEV * e_shard

    def body(x_ref, r_ref, w1_ref, w2_ref, out_ref,
             x_all, r_all, w_all, p_buf, c_buf,
             xs_s, xs_r, rs_s, rs_r, ws_s, ws_r, cs_s, cs_r):
        my = lax.axis_index("i")

        barrier = pltpu.get_barrier_semaphore()
        for d in (1, 2, 3):
            pl.semaphore_signal(
                barrier, inc=1, device_id=((my + d) % N_DEV,),
                device_id_type=pl.DeviceIdType.MESH)
        pl.semaphore_wait(barrier, N_DEV - 1)

        x_all[my, :, :] = x_ref[...].astype(jnp.bfloat16)
        r_all[my, :, :] = r_ref[...]

        sends = []
        for d in (1, 2, 3):
            peer = (my + d) % N_DEV
            xr = pltpu.make_async_remote_copy(
                src_ref=x_all.at[my], dst_ref=x_all.at[my],
                send_sem=xs_s.at[d], recv_sem=xs_r.at[d],
                device_id=(peer,), device_id_type=pl.DeviceIdType.MESH)
            xr.start()
            rr = pltpu.make_async_remote_copy(
                src_ref=r_all.at[my], dst_ref=r_all.at[my],
                send_sem=rs_s.at[d], recv_sem=rs_r.at[d],
                device_id=(peer,), device_id_type=pl.DeviceIdType.MESH)
            rr.start()
            sends += [xr, rr]

        w1b = [w1_ref[e, :, :].astype(jnp.bfloat16) for e in range(e_per)]
        w2b = [w2_ref[e, :, :].astype(jnp.bfloat16) for e in range(e_per)]

        for d in (1, 2, 3):
            src = (my - d) % N_DEV
            pltpu.make_async_remote_copy(
                src_ref=r_all.at[src], dst_ref=r_all.at[src],
                send_sem=rs_s.at[d], recv_sem=rs_r.at[d],
                device_id=(my,), device_id_type=pl.DeviceIdType.MESH
            ).wait_recv()
        rv = r_all[...]
        router_full = jnp.concatenate(
            [rv[0], rv[1], rv[2], rv[3]], axis=1)

        gates = jnp.dot(x_ref[...], router_full,
                        precision=lax.Precision.HIGHEST,
                        preferred_element_type=jnp.float32)
        eiota = lax.broadcasted_iota(jnp.int32, (t_per, n_exp), 1)
        v1 = jnp.max(gates, axis=1, keepdims=True)
        i1 = jnp.min(jnp.where(gates == v1, eiota, n_exp),
                     axis=1, keepdims=True)
        g2 = jnp.where(eiota == i1, -jnp.inf, gates)
        v2 = jnp.max(g2, axis=1, keepdims=True)
        i2 = jnp.min(jnp.where(g2 == v2, eiota, n_exp),
                     axis=1, keepdims=True)
        z = jnp.exp(v2 - v1)
        w_mine = (jnp.where(eiota == i1, 1.0, 0.0)
                  + jnp.where(eiota == i2, z, 0.0)) / (1.0 + z)

        w_all[my, :, :] = w_mine
        for d in (1, 2, 3):
            peer = (my + d) % N_DEV
            wr = pltpu.make_async_remote_copy(
                src_ref=w_all.at[my], dst_ref=w_all.at[my],
                send_sem=ws_s.at[d], recv_sem=ws_r.at[d],
                device_id=(peer,), device_id_type=pl.DeviceIdType.MESH)
            wr.start()
            sends.append(wr)

        def expert_slab(x_slab, w_rows):
            acc = jnp.zeros((t_per, d_model), jnp.float32)
            for e in range(e_per):
                h = jnp.dot(x_slab, w1b[e],
                            preferred_element_type=jnp.float32)
                h = jnp.maximum(h, 0.0).astype(jnp.bfloat16)
                o = jnp.dot(h, w2b[e],
                            preferred_element_type=jnp.float32)
                ge = e_shard * my + e
                wsel = jnp.sum(jnp.where(eiota == ge, w_rows, 0.0),
                               axis=1, keepdims=True)
                acc = acc + o * wsel
            return acc

        p_own = expert_slab(x_all[my, :, :], w_mine).astype(jnp.bfloat16)
        p_buf[my, :, :] = p_own
        c_buf[my, :, :] = p_own

        for d_x in (1, 3, 2):
            src = (my - d_x) % N_DEV
            pltpu.make_async_remote_copy(
                src_ref=x_all.at[src], dst_ref=x_all.at[src],
                send_sem=xs_s.at[d_x], recv_sem=xs_r.at[d_x],
                device_id=(my,), device_id_type=pl.DeviceIdType.MESH
            ).wait_recv()
            x_slab = x_all[src, :, :]
            pltpu.make_async_remote_copy(
                src_ref=w_all.at[src], dst_ref=w_all.at[src],
                send_sem=ws_s.at[d_x], recv_sem=ws_r.at[d_x],
                device_id=(my,), device_id_type=pl.DeviceIdType.MESH
            ).wait_recv()
            p_buf[src, :, :] = expert_slab(
                x_slab, w_all[src, :, :]).astype(jnp.bfloat16)
            d_c = (N_DEV - d_x) % N_DEV
            cr = pltpu.make_async_remote_copy(
                src_ref=p_buf.at[src], dst_ref=c_buf.at[my],
                send_sem=cs_s.at[d_c], recv_sem=cs_r.at[d_c],
                device_id=(src,), device_id_type=pl.DeviceIdType.MESH)
            cr.start()
            sends.append(cr)

        for d in (1, 2, 3):
            src = (my - d) % N_DEV
            pltpu.make_async_remote_copy(
                src_ref=c_buf.at[src], dst_ref=c_buf.at[src],
                send_sem=cs_s.at[d], recv_sem=cs_r.at[d],
                device_id=(my,), device_id_type=pl.DeviceIdType.MESH
            ).wait_recv()
        cv = c_buf[...]
        out_ref[...] = (cv[0].astype(jnp.float32)
                        + cv[1].astype(jnp.float32)
                        + cv[2].astype(jnp.float32)
                        + cv[3].astype(jnp.float32))

        for rd in sends:
            rd.wait_send()

    return pl.pallas_call(
        body,
        out_shape=jax.ShapeDtypeStruct((t_per, d_model), jnp.float32),
        in_specs=[pl.BlockSpec(memory_space=pltpu.VMEM)] * 4,
        out_specs=pl.BlockSpec(memory_space=pltpu.VMEM),
        scratch_shapes=[
            pltpu.VMEM((N_DEV, t_per, d_model), jnp.bfloat16),
            pltpu.VMEM((N_DEV, N_DEV * t_per, e_shard), jnp.float32),
            pltpu.VMEM((N_DEV, t_per, n_exp), jnp.float32),
            pltpu.VMEM((N_DEV, t_per, d_model), jnp.bfloat16),
            pltpu.VMEM((N_DEV, t_per, d_model), jnp.bfloat16),
            pltpu.SemaphoreType.DMA((N_DEV,)),
            pltpu.SemaphoreType.DMA((N_DEV,)),
            pltpu.SemaphoreType.DMA((N_DEV,)),
            pltpu.SemaphoreType.DMA((N_DEV,)),
            pltpu.SemaphoreType.DMA((N_DEV,)),
            pltpu.SemaphoreType.DMA((N_DEV,)),
            pltpu.SemaphoreType.DMA((N_DEV,)),
            pltpu.SemaphoreType.DMA((N_DEV,)),
        ],
        compiler_params=pltpu.CompilerParams(collective_id=0),
    )(x, router, W1, W2)
